# baseline (speedup 1.0000x reference)
import jax
import jax.numpy as jnp
from jax.experimental import pallas as pl

N_NODES = 10000

_W1_R, _W2_R, _W3_R, _WO1_R = 0, 128, 256, 384
_B1_R, _B2_R, _B3_R, _BO1_R, _WO2_R, _BO2_R = 512, 520, 528, 536, 544, 552
_PACK_ROWS = 560


def _fused_mlp_kernel(x_ref, p_ref, out_ref):
    h = jnp.dot(x_ref[...], p_ref[_W1_R:_W1_R + 128, :],
                preferred_element_type=jnp.float32)
    h = jnp.maximum(h + p_ref[_B1_R:_B1_R + 1, :], 0.0)
    h = jnp.dot(h, p_ref[_W2_R:_W2_R + 128, :],
                preferred_element_type=jnp.float32)
    h = jnp.maximum(h + p_ref[_B2_R:_B2_R + 1, :], 0.0)

    g = jnp.sum(h, axis=0, keepdims=True) * (1.0 / N_NODES)
    g = (jnp.dot(g, p_ref[_W3_R:_W3_R + 128, :],
                 preferred_element_type=jnp.float32)
         + p_ref[_B3_R:_B3_R + 1, :])
    q = (jnp.dot(g, p_ref[_WO1_R:_WO1_R + 128, :],
                 preferred_element_type=jnp.float32)
         + p_ref[_BO1_R:_BO1_R + 1, :])
    q = jnp.maximum(q, 0.0)
    out = jnp.sum(q * p_ref[_WO2_R:_WO2_R + 1, :]) + p_ref[_BO2_R, 0]
    out_ref[...] = out.reshape(1, 1)


def kernel(x, edge_index, W1, b1, W2, b2, W3, b3, Wo1, bo1, Wo2, bo2):
    del edge_index
    z7 = jnp.zeros((7, 128), jnp.float32)
    packed = jnp.concatenate([
        W1, W2, W3,
        jnp.pad(Wo1, ((0, 0), (0, 64))),
        b1.reshape(1, 128), z7,
        b2.reshape(1, 128), z7,
        b3.reshape(1, 128), z7,
        jnp.pad(bo1.reshape(1, 64), ((0, 0), (0, 64))), z7,
        jnp.pad(Wo2.reshape(1, 64), ((0, 0), (0, 64))), z7,
        jnp.pad(bo2.reshape(1, 1), ((0, 0), (0, 127))), z7,
    ], axis=0)

    out = pl.pallas_call(
        _fused_mlp_kernel,
        out_shape=jax.ShapeDtypeStruct((1, 1), jnp.float32),
    )(x, packed)
    return out


# speedup vs baseline: 1.7057x; 1.7057x over previous
import jax
import jax.numpy as jnp
from jax.experimental import pallas as pl
from jax.experimental.pallas import tpu as pltpu

N_NODES = 10000


def _fused_mlp_kernel(x_ref, W1_hbm, b1_hbm, W2_hbm, b2_hbm, W3_hbm, b3_hbm,
                      Wo1_hbm, bo1_hbm, Wo2_hbm, bo2_hbm, out_ref,
                      W1_v, b1_v, W2_v, b2_v, W3_v, b3_v,
                      Wo1_v, bo1_v, Wo2_v, bo2_v, sems):
    srcs = [W1_hbm, b1_hbm, W2_hbm, b2_hbm, W3_hbm, b3_hbm,
            Wo1_hbm, bo1_hbm, Wo2_hbm, bo2_hbm]
    dsts = [W1_v, b1_v, W2_v, b2_v, W3_v, b3_v, Wo1_v, bo1_v, Wo2_v, bo2_v]
    copies = [pltpu.make_async_copy(s, d, sems.at[i])
              for i, (s, d) in enumerate(zip(srcs, dsts))]
    for c in copies:
        c.start()

    copies[0].wait()
    copies[1].wait()
    h = jnp.dot(x_ref[...], W1_v[...], preferred_element_type=jnp.float32)
    h = jnp.maximum(h + b1_v[...], 0.0)

    copies[2].wait()
    copies[3].wait()
    h = jnp.dot(h, W2_v[...], preferred_element_type=jnp.float32)
    h = jnp.maximum(h + b2_v[...], 0.0)

    g = jnp.sum(h, axis=0, keepdims=True) * (1.0 / N_NODES)

    for c in copies[4:]:
        c.wait()
    g = jnp.dot(g, W3_v[...], preferred_element_type=jnp.float32) + b3_v[...]
    p = jnp.dot(g, Wo1_v[...], preferred_element_type=jnp.float32)
    p = jnp.maximum(p + bo1_v[...], 0.0)
    out_ref[...] = (jnp.dot(p, Wo2_v[...], preferred_element_type=jnp.float32)
                    + bo2_v[...])


def kernel(x, edge_index, W1, b1, W2, b2, W3, b3, Wo1, bo1, Wo2, bo2):
    del edge_index
    b1 = b1.reshape(1, -1)
    b2 = b2.reshape(1, -1)
    b3 = b3.reshape(1, -1)
    bo1 = bo1.reshape(1, -1)
    bo2 = bo2.reshape(1, -1)

    shapes = [W1, b1, W2, b2, W3, b3, Wo1, bo1, Wo2, bo2]
    out = pl.pallas_call(
        _fused_mlp_kernel,
        in_specs=[pl.BlockSpec(x.shape, lambda: (0, 0))]
        + [pl.BlockSpec(memory_space=pl.ANY)] * 10,
        out_specs=pl.BlockSpec((1, 1), lambda: (0, 0)),
        out_shape=jax.ShapeDtypeStruct((1, 1), jnp.float32),
        scratch_shapes=[pltpu.VMEM(a.shape, jnp.float32) for a in shapes]
        + [pltpu.SemaphoreType.DMA((10,))],
    )(x, W1, b1, W2, b2, W3, b3, Wo1, bo1, Wo2, bo2)
    return out
